# Initial kernel scaffold; baseline (speedup 1.0000x reference)
import jax
import jax.numpy as jnp
from jax.experimental import pallas as pl


def kernel(x, pos_emb):
    raise NotImplementedError("write your pallas kernel here")


# speedup vs baseline: 3.6062x; 3.6062x over previous
import jax
import jax.numpy as jnp
from jax.experimental import pallas as pl


_ROWS_PER_BLOCK = 256


def _copy_block(pos_emb_ref, out_ref):
    out_ref[...] = pos_emb_ref[...]


def kernel(x, pos_emb):
    seq_len = x.shape[1]
    d = pos_emb.shape[1]
    grid = (seq_len // _ROWS_PER_BLOCK,)
    out = pl.pallas_call(
        _copy_block,
        grid=grid,
        in_specs=[pl.BlockSpec((_ROWS_PER_BLOCK, d), lambda i: (i, 0))],
        out_specs=pl.BlockSpec((_ROWS_PER_BLOCK, d), lambda i: (i, 0)),
        out_shape=jax.ShapeDtypeStruct((seq_len, d), pos_emb.dtype),
    )(pos_emb[:seq_len])
    return out[None]
